# baseline (speedup 1.0000x reference)
import functools

import jax
import jax.numpy as jnp
from jax import lax
from jax.experimental import pallas as pl
from jax.experimental.pallas import tpu as pltpu
from jax.experimental.pallas import tpu_sc as plsc

VOCAB = 1000000
EMBED_DIM = 32
B = 16384
L = 50

NUM_CORES = 2
NUM_SUBCORES = 16
NW = NUM_CORES * NUM_SUBCORES
BW = B // NW
NT = BW // 128


def _make_kernel():
    mesh = plsc.VectorSubcoreMesh(core_axis_name="c", subcore_axis_name="s")

    @functools.partial(
        pl.kernel,
        mesh=mesh,
        out_type=jax.ShapeDtypeStruct((L, 4, 128, 8, 128), jnp.float32),
        scratch_types=[
            [pltpu.VMEM((BW,), jnp.int32) for _ in range(2)],
            [pltpu.VMEM((BW, EMBED_DIM), jnp.float32) for _ in range(2)],
            [pltpu.VMEM((4, NT, 8, 128), jnp.float32) for _ in range(2)],
            [pltpu.SemaphoreType.DMA for _ in range(2)],
            [pltpu.SemaphoreType.DMA for _ in range(2)],
        ],
        compiler_params=pltpu.CompilerParams(use_tc_tiling_on_sc=False,
                                             needs_layout_passes=False,
                                             disable_bounds_checks=True),
    )
    def gather_kernel(idx_hbm, table_hbm, out_hbm, idx_v, rows_v, stg_v,
                      gsem, wsem):
        wid = lax.axis_index("s") * NUM_CORES + lax.axis_index("c")
        b0 = wid * BW
        t0 = wid * NT
        lanes = lax.iota(jnp.int32, 16)

        rowvecs = [lanes + 16 * cb for cb in range(8)]

        def transpose_block(p):
            def tbody(d, carry):
                s = d >> 3
                r = d & 7
                col = jnp.broadcast_to(d, (16,))
                for tl in range(4):
                    for cb in range(8):
                        row = rowvecs[cb] + 128 * tl
                        v = plsc.load_gather(rows_v[p], [row, col])
                        stg_v[p][s, tl, r, pl.ds(16 * cb, 16)] = v
                return carry
            lax.fori_loop(0, 32, tbody, 0, unroll=4)

        def wb_copy(p, l):
            return pltpu.make_async_copy(
                stg_v[p], out_hbm.at[l, :, pl.ds(t0, NT)], wsem[p])

        def gather(p):
            return pltpu.async_copy(table_hbm.at[idx_v[p]], rows_v[p],
                                    gsem[p])

        def gather_wait(p):
            pltpu.make_async_copy(table_hbm.at[idx_v[p]], rows_v[p],
                                  gsem[p]).wait()

        pltpu.sync_copy(idx_hbm.at[0, pl.ds(b0, BW)], idx_v[0])
        gather(0)

        def body(j, carry):
            for p in range(2):
                l = 2 * j + p
                @pl.when(l < L - 1)
                def _():
                    pltpu.sync_copy(idx_hbm.at[l + 1, pl.ds(b0, BW)],
                                    idx_v[1 - p])
                gather_wait(p)

                @pl.when(l < L - 1)
                def _():
                    gather(1 - p)

                @pl.when(l >= 2)
                def _():
                    wb_copy(p, l - 2).wait()
                transpose_block(p)
                wb_copy(p, l).start()
            return carry

        lax.fori_loop(0, L // 2, body, 0)

        wb_copy(0, L - 2).wait()
        wb_copy(1, L - 1).wait()

    return gather_kernel


_gather = _make_kernel()


def kernel(words, table):
    out5 = _gather(words.T, table)
    return out5.transpose(2, 4, 0, 1, 3).reshape(B, L, EMBED_DIM)


# speedup vs baseline: 1.6975x; 1.6975x over previous
import functools

import jax
import jax.numpy as jnp
from jax import lax
from jax.experimental import pallas as pl
from jax.experimental.pallas import tpu as pltpu
from jax.experimental.pallas import tpu_sc as plsc

VOCAB = 1000000
EMBED_DIM = 32
B = 16384
L = 50

NUM_CORES = 2
NUM_SUBCORES = 16
NW = NUM_CORES * NUM_SUBCORES
BW = B // NW
NT = BW // 128


def _make_kernel():
    mesh = plsc.VectorSubcoreMesh(core_axis_name="c", subcore_axis_name="s")

    @functools.partial(
        pl.kernel,
        mesh=mesh,
        out_type=jax.ShapeDtypeStruct((L, 4, 128, 8, 128), jnp.float32),
        scratch_types=[
            [pltpu.VMEM((BW,), jnp.int32) for _ in range(2)],
            [pltpu.VMEM((BW, EMBED_DIM), jnp.float32) for _ in range(2)],
            [pltpu.VMEM((128, 129), jnp.float32) for _ in range(2)],
            [pltpu.SemaphoreType.DMA for _ in range(2)],
            [pltpu.SemaphoreType.DMA for _ in range(2)],
        ],
        compiler_params=pltpu.CompilerParams(use_tc_tiling_on_sc=False,
                                             needs_layout_passes=False,
                                             disable_bounds_checks=True),
    )
    def gather_kernel(idx_hbm, table_hbm, out_hbm, idx_v, rows_v, stg_v,
                      gsem, wsem):
        wid = lax.axis_index("s") * NUM_CORES + lax.axis_index("c")
        b0 = wid * BW
        t0 = wid * NT
        lanes = lax.iota(jnp.int32, 16)

        r0 = (lanes >> 3) * 32 + (lanes & 7)

        def transpose_block(p):
            def tbody(j, carry):
                tl8 = jnp.broadcast_to((j >> 7) * 8, (16,))
                colv = jnp.broadcast_to(j & 127, (16,))
                v0 = rows_v[p][j, pl.ds(0, 16)]
                v1 = rows_v[p][j, pl.ds(16, 16)]
                plsc.store_scatter(stg_v[p], [r0 + tl8, colv], v0)
                plsc.store_scatter(stg_v[p], [r0 + tl8 + 64, colv], v1)
                return carry
            lax.fori_loop(0, BW, tbody, 0, unroll=4)

        def wb_copy(p, l):
            class _WB:
                def start(self):
                    for m in range(16):
                        pltpu.async_copy(
                            stg_v[p].at[pl.ds(8 * m, 8), pl.ds(0, 128)],
                            out_hbm.at[l, m // 4, t0 + m % 4], wsem[p])
                def wait(self):
                    for m in range(16):
                        pltpu.make_async_copy(
                            stg_v[p].at[pl.ds(8 * m, 8), pl.ds(0, 128)],
                            out_hbm.at[l, m // 4, t0 + m % 4],
                            wsem[p]).wait()
            return _WB()

        def gather(p):
            return pltpu.async_copy(table_hbm.at[idx_v[p]], rows_v[p],
                                    gsem[p])

        def gather_wait(p):
            pltpu.make_async_copy(table_hbm.at[idx_v[p]], rows_v[p],
                                  gsem[p]).wait()

        pltpu.sync_copy(idx_hbm.at[0, pl.ds(b0, BW)], idx_v[0])
        gather(0)

        def body(j, carry):
            for p in range(2):
                l = 2 * j + p
                @pl.when(l < L - 1)
                def _():
                    pltpu.sync_copy(idx_hbm.at[l + 1, pl.ds(b0, BW)],
                                    idx_v[1 - p])
                gather_wait(p)

                @pl.when(l < L - 1)
                def _():
                    gather(1 - p)

                @pl.when(l >= 2)
                def _():
                    wb_copy(p, l - 2).wait()
                transpose_block(p)
                wb_copy(p, l).start()
            return carry

        lax.fori_loop(0, L // 2, body, 0)

        wb_copy(0, L - 2).wait()
        wb_copy(1, L - 1).wait()

    return gather_kernel


_gather = _make_kernel()


def kernel(words, table):
    out5 = _gather(words.T, table)
    return out5.transpose(2, 4, 0, 1, 3).reshape(B, L, EMBED_DIM)


# speedup vs baseline: 1.6984x; 1.0006x over previous
import functools

import jax
import jax.numpy as jnp
from jax import lax
from jax.experimental import pallas as pl
from jax.experimental.pallas import tpu as pltpu
from jax.experimental.pallas import tpu_sc as plsc

VOCAB = 1000000
EMBED_DIM = 32
B = 16384
L = 50

NUM_CORES = 2
NUM_SUBCORES = 16
NW = NUM_CORES * NUM_SUBCORES
BW = B // NW
NT = BW // 128


def _make_kernel():
    mesh = plsc.VectorSubcoreMesh(core_axis_name="c", subcore_axis_name="s")

    @functools.partial(
        pl.kernel,
        mesh=mesh,
        out_type=jax.ShapeDtypeStruct((L, 4, 128, 8, 128), jnp.float32),
        scratch_types=[
            [pltpu.VMEM((BW,), jnp.int32) for _ in range(2)],
            [pltpu.VMEM((BW, EMBED_DIM), jnp.float32) for _ in range(2)],
            [pltpu.VMEM((128, 129), jnp.float32) for _ in range(2)],
            [pltpu.SemaphoreType.DMA for _ in range(2)],
            [pltpu.SemaphoreType.DMA for _ in range(2)],
        ],
        compiler_params=pltpu.CompilerParams(use_tc_tiling_on_sc=False,
                                             needs_layout_passes=False,
                                             disable_bounds_checks=True),
    )
    def gather_kernel(idx_hbm, table_hbm, out_hbm, idx_v, rows_v, stg_v,
                      gsem, wsem):
        wid = lax.axis_index("s") * NUM_CORES + lax.axis_index("c")
        b0 = wid * BW
        t0 = wid * NT
        lanes = lax.iota(jnp.int32, 16)

        r0 = (lanes >> 3) * 32 + (lanes & 7)

        def transpose_block(p):
            def tbody(j, carry):
                tl8 = jnp.broadcast_to((j >> 7) * 8, (16,))
                colv = jnp.broadcast_to(j & 127, (16,))
                v0 = rows_v[p][j, pl.ds(0, 16)]
                v1 = rows_v[p][j, pl.ds(16, 16)]
                plsc.store_scatter(stg_v[p], [r0 + tl8, colv], v0)
                plsc.store_scatter(stg_v[p], [r0 + tl8 + 64, colv], v1)
                return carry
            lax.fori_loop(0, BW, tbody, 0, unroll=8)

        def wb_copy(p, l):
            class _WB:
                def start(self):
                    for m in range(16):
                        pltpu.async_copy(
                            stg_v[p].at[pl.ds(8 * m, 8), pl.ds(0, 128)],
                            out_hbm.at[l, m // 4, t0 + m % 4], wsem[p])
                def wait(self):
                    for m in range(16):
                        pltpu.make_async_copy(
                            stg_v[p].at[pl.ds(8 * m, 8), pl.ds(0, 128)],
                            out_hbm.at[l, m // 4, t0 + m % 4],
                            wsem[p]).wait()
            return _WB()

        def gather(p):
            return pltpu.async_copy(table_hbm.at[idx_v[p]], rows_v[p],
                                    gsem[p])

        def gather_wait(p):
            pltpu.make_async_copy(table_hbm.at[idx_v[p]], rows_v[p],
                                  gsem[p]).wait()

        pltpu.sync_copy(idx_hbm.at[0, pl.ds(b0, BW)], idx_v[0])
        gather(0)

        def body(j, carry):
            for p in range(2):
                l = 2 * j + p
                @pl.when(l < L - 1)
                def _():
                    pltpu.sync_copy(idx_hbm.at[l + 1, pl.ds(b0, BW)],
                                    idx_v[1 - p])
                gather_wait(p)

                @pl.when(l < L - 1)
                def _():
                    gather(1 - p)

                @pl.when(l >= 2)
                def _():
                    wb_copy(p, l - 2).wait()
                transpose_block(p)
                wb_copy(p, l).start()
            return carry

        lax.fori_loop(0, L // 2, body, 0)

        wb_copy(0, L - 2).wait()
        wb_copy(1, L - 1).wait()

    return gather_kernel


_gather = _make_kernel()


def kernel(words, table):
    out5 = _gather(words.T, table)
    return out5.transpose(2, 4, 0, 1, 3).reshape(B, L, EMBED_DIM)
